# baseline (speedup 1.0000x reference)
import functools

import jax
import jax.numpy as jnp
from jax import lax
from jax.experimental import pallas as pl
from jax.experimental.pallas import tpu as pltpu
from jax.experimental.pallas import tpu_sc as plsc

N = 10000
E = 320000
D_IN = 128
D_H = 128
D_OUT = 64

NC = 2
NS = 16
NW = NC * NS
CHUNK = 128
CPW = 80
E_PAD = NW * CPW * CHUNK
N_PAD = 10112
RPT_PAD = N_PAD // NS

_mesh = plsc.VectorSubcoreMesh(core_axis_name="c", subcore_axis_name="s")


D_DEG = 16

@functools.partial(
    pl.kernel,
    mesh=_mesh,
    out_type=jax.ShapeDtypeStruct((NC, N_PAD, D_DEG), jnp.float32),
    scratch_types=[
        pltpu.VMEM_SHARED((N_PAD, D_DEG), jnp.float32),
        pltpu.VMEM((CPW, CHUNK), jnp.int32),
        pltpu.VMEM((CHUNK, D_DEG), jnp.float32),
        pltpu.SemaphoreType.DMA,
    ],
)
def _hist_kernel(idx_hbm, ones_hbm, zeros_hbm, deg_hbm, sh, idx_v, ones_v,
                 ssem):
    c = lax.axis_index("c")
    s = lax.axis_index("s")
    wid = c * NS + s
    pltpu.sync_copy(idx_hbm.at[wid], idx_v)
    pltpu.sync_copy(ones_hbm, ones_v)
    z0 = s * RPT_PAD
    pltpu.sync_copy(zeros_hbm, sh.at[pl.ds(z0, RPT_PAD)])
    plsc.subcore_barrier()

    def fire(j, carry):
        pltpu.async_copy(ones_v, sh.at[idx_v.at[j]], ssem, add=True)
        return carry

    lax.fori_loop(0, CPW, fire, 0)

    def drain(j, carry):
        pltpu.make_async_copy(ones_v, sh.at[idx_v.at[j]], ssem).wait()
        return carry

    lax.fori_loop(0, CPW, drain, 0)
    plsc.subcore_barrier()
    pltpu.sync_copy(sh.at[pl.ds(z0, RPT_PAD)],
                    deg_hbm.at[c, pl.ds(z0, RPT_PAD)])


def _make_agg(D):
    @functools.partial(
        pl.kernel,
        mesh=_mesh,
        out_type=jax.ShapeDtypeStruct((NC, N_PAD, D), jnp.float32),
        scratch_types=[
            pltpu.VMEM_SHARED((N_PAD, D), jnp.float32),
            pltpu.VMEM((CPW, CHUNK), jnp.int32),
            pltpu.VMEM((CPW, CHUNK), jnp.int32),
            pltpu.VMEM((CHUNK, D), jnp.float32),
            pltpu.SemaphoreType.DMA,
        ],
    )
    def _agg(hw_hbm, src_hbm, dst_hbm, zeros_hbm, out_hbm, sh, isrc, idst,
             buf, gsem):
        c = lax.axis_index("c")
        s = lax.axis_index("s")
        wid = c * NS + s
        pltpu.sync_copy(src_hbm.at[wid], isrc)
        pltpu.sync_copy(dst_hbm.at[wid], idst)
        z0 = s * RPT_PAD
        pltpu.sync_copy(zeros_hbm, sh.at[pl.ds(z0, RPT_PAD)])
        plsc.subcore_barrier()

        def body(j, carry):
            pltpu.async_copy(hw_hbm.at[isrc.at[j]], buf, gsem).wait()
            pltpu.sync_copy(buf, sh.at[idst.at[j]], add=True)
            return carry

        lax.fori_loop(0, CPW, body, 0)
        plsc.subcore_barrier()
        pltpu.sync_copy(sh.at[pl.ds(z0, RPT_PAD)],
                        out_hbm.at[c, pl.ds(z0, RPT_PAD)])

    return _agg


_agg128 = _make_agg(D_H)


def _tc_first_body(f_ref, w_ref, ns_ref, hw_ref):
    hw_ref[...] = jnp.dot(f_ref[...], w_ref[...],
                          preferred_element_type=jnp.float32) * ns_ref[...]


_tc_first = pl.pallas_call(
    _tc_first_body,
    out_shape=jax.ShapeDtypeStruct((N, D_H), jnp.float32),
)


def _tc_mid_body(p0_ref, p1_ref, nd_ref, ns_ref, b_ref, w_ref, o_ref):
    h = (p0_ref[...] + p1_ref[...]) * nd_ref[...] + b_ref[...]
    h = jnp.maximum(h, 0.0)
    o_ref[...] = jnp.dot(h, w_ref[...],
                         preferred_element_type=jnp.float32) * ns_ref[...]


_tc_mid = pl.pallas_call(
    _tc_mid_body,
    out_shape=jax.ShapeDtypeStruct((N_PAD, D_H), jnp.float32),
)


def _tc_last_body(p0_ref, p1_ref, nd_ref, b_ref, o_ref):
    agg = p0_ref[:, :D_OUT] + p1_ref[:, :D_OUT]
    o_ref[...] = agg * nd_ref[...] + b_ref[...]


_tc_last = pl.pallas_call(
    _tc_last_body,
    out_shape=jax.ShapeDtypeStruct((N_PAD, D_OUT), jnp.float32),
)


@jax.jit
def kernel(features, edge_index, W0, b0, W1, b1, W2, b2):
    src = edge_index[0]
    dst = edge_index[1]
    pad = E_PAD - E
    trash = N + (jnp.arange(pad, dtype=jnp.int32) % (N_PAD - N))
    spread = jnp.arange(pad, dtype=jnp.int32) % N
    src_deg = jnp.concatenate([src, trash]).reshape(NW, CPW, CHUNK)
    dst_any = jnp.concatenate([dst, trash]).reshape(NW, CPW, CHUNK)
    src_gat = jnp.concatenate([src, spread]).reshape(NW, CPW, CHUNK)

    ones16 = jnp.ones((CHUNK, D_DEG), jnp.float32)
    zer16 = jnp.zeros((RPT_PAD, D_DEG), jnp.float32)
    zer128 = jnp.zeros((RPT_PAD, D_H), jnp.float32)
    W2p = jnp.pad(W2, ((0, 0), (0, D_H - D_OUT)))

    dego = _hist_kernel(src_deg, ones16, zer16)
    degi = _hist_kernel(dst_any, ones16, zer16)
    deg_o = dego[0, :, 0:1] + dego[1, :, 0:1]
    deg_i = degi[0, :, 0:1] + degi[1, :, 0:1]
    ns_pad = jnp.where(deg_o > 0.0, lax.rsqrt(jnp.maximum(deg_o, 1.0)), 0.0)
    nd_pad = jnp.where(deg_i > 0.0, lax.rsqrt(jnp.maximum(deg_i, 1.0)), 0.0)

    hw0 = _tc_first(features, W0, ns_pad[:N])
    p0 = _agg128(hw0, src_gat, dst_any, zer128)
    hw1 = _tc_mid(p0[0], p0[1], nd_pad, ns_pad, b0.reshape(1, D_H), W1)
    p1 = _agg128(hw1, src_gat, dst_any, zer128)
    hw2 = _tc_mid(p1[0], p1[1], nd_pad, ns_pad, b1.reshape(1, D_H), W2p)
    p2 = _agg128(hw2, src_gat, dst_any, zer128)
    out = _tc_last(p2[0], p2[1], nd_pad, b2.reshape(1, D_OUT))
    return out[:N]


# speedup vs baseline: 1.7773x; 1.7773x over previous
import functools

import jax
import jax.numpy as jnp
from jax import lax
from jax.experimental import pallas as pl
from jax.experimental.pallas import tpu as pltpu
from jax.experimental.pallas import tpu_sc as plsc

N = 10000
E = 320000
D_IN = 128
D_H = 128
D_OUT = 64

NC = 2
NS = 16
NW = NC * NS
CHUNK = 128
CPW = 80
E_PAD = NW * CPW * CHUNK
N_PAD = 10112
RPT_PAD = N_PAD // NS

_mesh = plsc.VectorSubcoreMesh(core_axis_name="c", subcore_axis_name="s")


@functools.partial(
    pl.kernel,
    mesh=_mesh,
    out_type=jax.ShapeDtypeStruct((NC, N_PAD, D_H), jnp.float32),
    scratch_types=[
        pltpu.VMEM_SHARED((N_PAD, D_H), jnp.float32),
        pltpu.VMEM((CPW, CHUNK), jnp.int32),
        pltpu.VMEM((CHUNK, D_H), jnp.float32),
        pltpu.SemaphoreType.DMA,
    ],
)
def _hist_kernel(idx_hbm, ones_hbm, zeros_hbm, deg_hbm, sh, idx_v, ones_v,
                 ssem):
    c = lax.axis_index("c")
    s = lax.axis_index("s")
    wid = c * NS + s
    pltpu.sync_copy(idx_hbm.at[wid], idx_v)
    pltpu.sync_copy(ones_hbm, ones_v)
    z0 = s * RPT_PAD
    pltpu.sync_copy(zeros_hbm, sh.at[pl.ds(z0, RPT_PAD)])
    plsc.subcore_barrier()

    def fire(j, carry):
        pltpu.async_copy(ones_v, sh.at[idx_v.at[j]], ssem, add=True)
        return carry

    lax.fori_loop(0, CPW, fire, 0)

    def drain(j, carry):
        pltpu.make_async_copy(ones_v, sh.at[idx_v.at[j]], ssem).wait()
        return carry

    lax.fori_loop(0, CPW, drain, 0)
    plsc.subcore_barrier()
    pltpu.sync_copy(sh.at[pl.ds(z0, RPT_PAD)],
                    deg_hbm.at[c, pl.ds(z0, RPT_PAD)])


def _make_agg(D):
    HCPW = CPW // 2

    @functools.partial(
        pl.kernel,
        mesh=_mesh,
        out_type=jax.ShapeDtypeStruct((NC, N_PAD, D), jnp.float32),
        scratch_types=[
            pltpu.VMEM_SHARED((N_PAD, D), jnp.float32),
            pltpu.VMEM((HCPW, CHUNK), jnp.int32),
            pltpu.VMEM((HCPW, CHUNK), jnp.int32),
            pltpu.VMEM((CHUNK, D), jnp.float32),
            pltpu.VMEM((CHUNK, D), jnp.float32),
            pltpu.SemaphoreType.DMA,
            pltpu.SemaphoreType.DMA,
        ],
    )
    def _agg(hw_hbm, src_hbm, dst_hbm, zeros_hbm, out_hbm, sh, isrc, idst,
             buf0, buf1, sem0, sem1):
        c = lax.axis_index("c")
        s = lax.axis_index("s")
        wid = c * NS + s
        z0 = s * RPT_PAD
        pltpu.sync_copy(zeros_hbm, sh.at[pl.ds(z0, RPT_PAD)])
        plsc.subcore_barrier()

        bufs = (buf0, buf1)
        sems = (sem0, sem1)

        def half(h, carry):
            pltpu.sync_copy(src_hbm.at[wid, pl.ds(h * HCPW, HCPW)], isrc)
            pltpu.sync_copy(dst_hbm.at[wid, pl.ds(h * HCPW, HCPW)], idst)

            def step(jj, carry):
                for b in range(2):
                    j = jj * 2 + b

                    @pl.when(j < HCPW)
                    def _fire(j=j, b=b):
                        pltpu.async_copy(hw_hbm.at[isrc.at[j]], bufs[b],
                                         sems[b])

                    @pl.when(jnp.logical_and(j >= 1, j <= HCPW))
                    def _scat(j=j, b=b):
                        pltpu.make_async_copy(hw_hbm.at[isrc.at[j - 1]],
                                              bufs[1 - b], sems[1 - b]).wait()
                        pltpu.sync_copy(bufs[1 - b], sh.at[idst.at[j - 1]],
                                        add=True)
                return carry

            lax.fori_loop(0, HCPW // 2 + 1, step, 0)
            return carry

        lax.fori_loop(0, 2, half, 0)
        plsc.subcore_barrier()
        pltpu.sync_copy(sh.at[pl.ds(z0, RPT_PAD)],
                        out_hbm.at[c, pl.ds(z0, RPT_PAD)])

    return _agg


_agg128 = _make_agg(D_H)


def _tc_first_body(f_ref, w_ref, ns_ref, hw_ref):
    hw_ref[...] = jnp.dot(f_ref[...], w_ref[...],
                          preferred_element_type=jnp.float32) * ns_ref[...]


_tc_first = pl.pallas_call(
    _tc_first_body,
    out_shape=jax.ShapeDtypeStruct((N, D_H), jnp.float32),
)


def _tc_mid_body(p0_ref, p1_ref, nd_ref, ns_ref, b_ref, w_ref, o_ref):
    h = (p0_ref[...] + p1_ref[...]) * nd_ref[...] + b_ref[...]
    h = jnp.maximum(h, 0.0)
    o_ref[...] = jnp.dot(h, w_ref[...],
                         preferred_element_type=jnp.float32) * ns_ref[...]


_tc_mid = pl.pallas_call(
    _tc_mid_body,
    out_shape=jax.ShapeDtypeStruct((N_PAD, D_H), jnp.float32),
)


def _tc_last_body(p0_ref, p1_ref, nd_ref, b_ref, o_ref):
    agg = p0_ref[:, :D_OUT] + p1_ref[:, :D_OUT]
    o_ref[...] = agg * nd_ref[...] + b_ref[...]


_tc_last = pl.pallas_call(
    _tc_last_body,
    out_shape=jax.ShapeDtypeStruct((N_PAD, D_OUT), jnp.float32),
)


@jax.jit
def kernel(features, edge_index, W0, b0, W1, b1, W2, b2):
    src = edge_index[0]
    dst = edge_index[1]
    pad = E_PAD - E
    trash = N + (jnp.arange(pad, dtype=jnp.int32) % (N_PAD - N))
    spread = jnp.arange(pad, dtype=jnp.int32) % N
    src_deg = jnp.concatenate([src, trash]).reshape(NW, CPW, CHUNK)
    dst_any = jnp.concatenate([dst, trash]).reshape(NW, CPW, CHUNK)
    src_gat = jnp.concatenate([src, spread]).reshape(NW, CPW, CHUNK)

    ones128 = jnp.ones((CHUNK, D_H), jnp.float32)
    zer128 = jnp.zeros((RPT_PAD, D_H), jnp.float32)
    W2p = jnp.pad(W2, ((0, 0), (0, D_H - D_OUT)))

    dego = _hist_kernel(src_deg, ones128, zer128)
    degi = _hist_kernel(dst_any, ones128, zer128)
    deg_o = dego[0, :, 0:1] + dego[1, :, 0:1]
    deg_i = degi[0, :, 0:1] + degi[1, :, 0:1]
    ns_pad = jnp.where(deg_o > 0.0, lax.rsqrt(jnp.maximum(deg_o, 1.0)), 0.0)
    nd_pad = jnp.where(deg_i > 0.0, lax.rsqrt(jnp.maximum(deg_i, 1.0)), 0.0)

    hw0 = _tc_first(features, W0, ns_pad[:N])
    p0 = _agg128(hw0, src_gat, dst_any, zer128)
    hw1 = _tc_mid(p0[0], p0[1], nd_pad, ns_pad, b0.reshape(1, D_H), W1)
    p1 = _agg128(hw1, src_gat, dst_any, zer128)
    hw2 = _tc_mid(p1[0], p1[1], nd_pad, ns_pad, b1.reshape(1, D_H), W2p)
    p2 = _agg128(hw2, src_gat, dst_any, zer128)
    out = _tc_last(p2[0], p2[1], nd_pad, b2.reshape(1, D_OUT))
    return out[:N]
